# baseline (speedup 1.0000x reference)
import functools

import jax
import jax.numpy as jnp
from jax import lax
from jax.experimental import pallas as pl
from jax.experimental.pallas import tpu as pltpu
from jax.experimental.pallas import tpu_sc as plsc

N = 10000
E = 320000
D = 128
NC = 2
NS = 16
NW = NC * NS
EPW = E // NW
CH = 80
NCH = EPW // CH
NP = 10112
RPT = NP // NS

_SC_MESH = plsc.VectorSubcoreMesh(core_axis_name="c", subcore_axis_name="s")


@functools.partial(
    pl.kernel,
    out_type=jax.ShapeDtypeStruct((NC, NP, D), jnp.float32),
    mesh=_SC_MESH,
    scratch_types=[
        pltpu.VMEM((NCH, CH), jnp.int32),
        pltpu.VMEM((CH, D), jnp.float32),
        pltpu.VMEM_SHARED((NP, D), jnp.float32),
    ],
)
def _sc_degree(dst_hbm, ones_hbm, zeros_hbm, out_hbm, dst_v, ones_v, accum):
    c = lax.axis_index("c")
    s = lax.axis_index("s")
    w = s * NC + c
    pltpu.sync_copy(dst_hbm.at[w], dst_v)
    pltpu.sync_copy(ones_hbm, ones_v)
    pltpu.sync_copy(zeros_hbm, accum.at[pl.ds(s * RPT, RPT)])
    plsc.subcore_barrier()

    def body(j, carry):
        pltpu.sync_copy(ones_v, accum.at[dst_v.at[j]], add=True)
        return carry

    lax.fori_loop(0, NCH, body, 0)
    plsc.subcore_barrier()
    pltpu.sync_copy(accum.at[pl.ds(s * RPT, RPT)], out_hbm.at[c, pl.ds(s * RPT, RPT)])


@functools.partial(
    pl.kernel,
    out_type=jax.ShapeDtypeStruct((NC, NP, D), jnp.float32),
    mesh=_SC_MESH,
    scratch_types=[
        pltpu.VMEM((NCH, CH), jnp.int32),
        pltpu.VMEM((NCH, CH), jnp.int32),
        pltpu.VMEM((CH, D), jnp.float32),
        pltpu.VMEM_SHARED((NP, D), jnp.float32),
        pltpu.SemaphoreType.DMA,
    ],
)
def _sc_spmm(y_hbm, src_hbm, dst_hbm, zeros_hbm, out_hbm,
             src_v, dst_v, buf, accum, sem):
    c = lax.axis_index("c")
    s = lax.axis_index("s")
    w = s * NC + c
    pltpu.sync_copy(src_hbm.at[w], src_v)
    pltpu.sync_copy(dst_hbm.at[w], dst_v)
    pltpu.sync_copy(zeros_hbm, accum.at[pl.ds(s * RPT, RPT)])
    plsc.subcore_barrier()

    def body(j, carry):
        pltpu.async_copy(y_hbm.at[src_v.at[j]], buf, sem).wait()
        pltpu.sync_copy(buf, accum.at[dst_v.at[j]], add=True)
        return carry

    lax.fori_loop(0, NCH, body, 0)
    plsc.subcore_barrier()
    pltpu.sync_copy(accum.at[pl.ds(s * RPT, RPT)], out_hbm.at[c, pl.ds(s * RPT, RPT)])


def _tc_matmul_body(x_ref, w_ref, out_ref):
    out_ref[...] = jnp.dot(x_ref[...], w_ref[...], preferred_element_type=jnp.float32)


def _tc_scale_body(degp_ref, xw_ref, y_ref, dinv_ref):
    indeg = degp_ref[0, :N, 0:1] + degp_ref[1, :N, 0:1]
    dinv = lax.rsqrt(indeg + 1.0)
    y_ref[...] = xw_ref[...] * dinv
    dinv_ref[...] = dinv


def _tc_mid_body(part_ref, y_ref, dinv_ref, b_ref, g_ref, be_ref, w_ref, out_ref):
    dinv = dinv_ref[...]
    h = (part_ref[0, :N] + part_ref[1, :N] + y_ref[...]) * dinv + b_ref[...]
    m = jnp.mean(h, axis=0)
    v = jnp.mean((h - m) ** 2, axis=0)
    h = jnp.maximum(g_ref[...] * (h - m) * lax.rsqrt(v + 1e-5) + be_ref[...], 0.0)
    out_ref[...] = jnp.dot(h, w_ref[...], preferred_element_type=jnp.float32) * dinv


def _tc_head_body(part_ref, y_ref, dinv_ref, b_ref, g_ref, be_ref,
                  wh_ref, bh_ref, g3_ref, be3_ref, wo_ref, bo_ref, out_ref):
    dinv = dinv_ref[...]
    h = (part_ref[0, :N] + part_ref[1, :N] + y_ref[...]) * dinv + b_ref[...]
    m = jnp.mean(h, axis=0)
    v = jnp.mean((h - m) ** 2, axis=0)
    h = jnp.maximum(g_ref[...] * (h - m) * lax.rsqrt(v + 1e-5) + be_ref[...], 0.0)
    h3 = jnp.maximum(
        jnp.dot(h, wh_ref[...], preferred_element_type=jnp.float32) + bh_ref[...], 0.0)
    m3 = jnp.mean(h3, axis=0)
    v3 = jnp.mean((h3 - m3) ** 2, axis=0)
    h3 = g3_ref[...] * (h3 - m3) * lax.rsqrt(v3 + 1e-5) + be3_ref[...]
    logits = jnp.dot(h3, wo_ref[...], preferred_element_type=jnp.float32) + bo_ref[...]
    col = lax.broadcasted_iota(jnp.int32, logits.shape, 1)
    masked = jnp.where(col < 10, logits, -1e30)
    mx = jnp.max(masked, axis=1, keepdims=True)
    lse = jnp.log(jnp.sum(jnp.exp(masked - mx), axis=1, keepdims=True)) + mx
    out_ref[...] = masked - lse


def kernel(x, edge_index, batch, W1, b1, g1, be1, W2, b2, g2, be2, Wh, bh, g3, be3, Wo, bo):
    src3 = edge_index[0].reshape(NW, NCH, CH)
    dst3 = edge_index[1].reshape(NW, NCH, CH)
    onesD = jnp.ones((CH, D), jnp.float32)
    zerosD = jnp.zeros((RPT, D), jnp.float32)

    degp = _sc_degree(dst3, onesD, zerosD)

    xw1 = pl.pallas_call(
        _tc_matmul_body,
        out_shape=jax.ShapeDtypeStruct((N, D), jnp.float32),
    )(x, W1)

    y1, dinv = pl.pallas_call(
        _tc_scale_body,
        out_shape=[jax.ShapeDtypeStruct((N, D), jnp.float32),
                   jax.ShapeDtypeStruct((N, 1), jnp.float32)],
    )(degp, xw1)

    part1 = _sc_spmm(y1, src3, dst3, zerosD)

    y2 = pl.pallas_call(
        _tc_mid_body,
        out_shape=jax.ShapeDtypeStruct((N, D), jnp.float32),
    )(part1, y1, dinv, b1.reshape(1, D), g1.reshape(1, D), be1.reshape(1, D), W2)

    part2 = _sc_spmm(y2, src3, dst3, zerosD)

    out_pad = pl.pallas_call(
        _tc_head_body,
        out_shape=jax.ShapeDtypeStruct((N, D), jnp.float32),
    )(part2, y2, dinv, b2.reshape(1, D), g2.reshape(1, D), be2.reshape(1, D),
      Wh, bh.reshape(1, 64), g3.reshape(1, 64), be3.reshape(1, 64),
      jnp.pad(Wo, ((0, 0), (0, D - 10))), jnp.pad(bo, (0, D - 10)).reshape(1, D))
    return out_pad[:, :10]


# speedup vs baseline: 1.0894x; 1.0894x over previous
import functools

import jax
import jax.numpy as jnp
from jax import lax
from jax.experimental import pallas as pl
from jax.experimental.pallas import tpu as pltpu
from jax.experimental.pallas import tpu_sc as plsc

N = 10000
E = 320000
D = 128
NC = 2
NS = 16
NW = NC * NS
EPW = E // NW
CH = 80
NCH = EPW // CH
NP = 10112
RPT = NP // NS

_SC_MESH = plsc.VectorSubcoreMesh(core_axis_name="c", subcore_axis_name="s")


@functools.partial(
    pl.kernel,
    out_type=jax.ShapeDtypeStruct((NC, NP, 16), jnp.float32),
    mesh=_SC_MESH,
    scratch_types=[
        pltpu.VMEM((NCH, CH), jnp.int32),
        pltpu.VMEM((CH, 16), jnp.float32),
        pltpu.VMEM_SHARED((NP, 16), jnp.float32),
    ],
    compiler_params=pltpu.CompilerParams(use_tc_tiling_on_sc=False),
)
def _sc_degree(dst_hbm, ones_hbm, zeros_hbm, out_hbm, dst_v, ones_v, accum):
    c = lax.axis_index("c")
    s = lax.axis_index("s")
    w = s * NC + c
    pltpu.sync_copy(dst_hbm.at[w], dst_v)
    pltpu.sync_copy(ones_hbm, ones_v)
    pltpu.sync_copy(zeros_hbm, accum.at[pl.ds(s * RPT, RPT)])
    plsc.subcore_barrier()

    def body(j, carry):
        pltpu.sync_copy(ones_v, accum.at[dst_v.at[j]], add=True)
        return carry

    lax.fori_loop(0, NCH, body, 0)
    plsc.subcore_barrier()
    pltpu.sync_copy(accum.at[pl.ds(s * RPT, RPT)], out_hbm.at[c, pl.ds(s * RPT, RPT)])


@functools.partial(
    pl.kernel,
    out_type=jax.ShapeDtypeStruct((NC, NP, D), jnp.float32),
    mesh=_SC_MESH,
    scratch_types=[
        pltpu.VMEM((NCH, CH), jnp.int32),
        pltpu.VMEM((NCH, CH), jnp.int32),
        pltpu.VMEM((CH, D), jnp.float32),
        pltpu.VMEM_SHARED((NP, D), jnp.float32),
        pltpu.SemaphoreType.DMA,
    ],
)
def _sc_spmm(y_hbm, src_hbm, dst_hbm, zeros_hbm, out_hbm,
             src_v, dst_v, buf, accum, sem):
    c = lax.axis_index("c")
    s = lax.axis_index("s")
    w = s * NC + c
    pltpu.sync_copy(src_hbm.at[w], src_v)
    pltpu.sync_copy(dst_hbm.at[w], dst_v)
    pltpu.sync_copy(zeros_hbm, accum.at[pl.ds(s * RPT, RPT)])
    plsc.subcore_barrier()

    def body(j, carry):
        pltpu.async_copy(y_hbm.at[src_v.at[j]], buf, sem).wait()
        pltpu.sync_copy(buf, accum.at[dst_v.at[j]], add=True)
        return carry

    lax.fori_loop(0, NCH, body, 0)
    plsc.subcore_barrier()
    pltpu.sync_copy(accum.at[pl.ds(s * RPT, RPT)], out_hbm.at[c, pl.ds(s * RPT, RPT)])


def _tc_first_body(degp_ref, x_ref, w_ref, y_ref, dinv_ref):
    indeg = degp_ref[0, :N, 0:1] + degp_ref[1, :N, 0:1]
    dinv = lax.rsqrt(indeg + 1.0)
    xw = jnp.dot(x_ref[...], w_ref[...], preferred_element_type=jnp.float32)
    y_ref[...] = xw * dinv
    dinv_ref[...] = dinv


def _tc_mid_body(part_ref, y_ref, dinv_ref, b_ref, g_ref, be_ref, w_ref, out_ref):
    dinv = dinv_ref[...]
    h = (part_ref[0, :N] + part_ref[1, :N] + y_ref[...]) * dinv + b_ref[...]
    m = jnp.mean(h, axis=0)
    v = jnp.mean((h - m) ** 2, axis=0)
    h = jnp.maximum(g_ref[...] * (h - m) * lax.rsqrt(v + 1e-5) + be_ref[...], 0.0)
    out_ref[...] = jnp.dot(h, w_ref[...], preferred_element_type=jnp.float32) * dinv


def _tc_head_body(part_ref, y_ref, dinv_ref, b_ref, g_ref, be_ref,
                  wh_ref, bh_ref, g3_ref, be3_ref, wo_ref, bo_ref, out_ref):
    dinv = dinv_ref[...]
    h = (part_ref[0, :N] + part_ref[1, :N] + y_ref[...]) * dinv + b_ref[...]
    m = jnp.mean(h, axis=0)
    v = jnp.mean((h - m) ** 2, axis=0)
    h = jnp.maximum(g_ref[...] * (h - m) * lax.rsqrt(v + 1e-5) + be_ref[...], 0.0)
    h3 = jnp.maximum(
        jnp.dot(h, wh_ref[...], preferred_element_type=jnp.float32) + bh_ref[...], 0.0)
    m3 = jnp.mean(h3, axis=0)
    v3 = jnp.mean((h3 - m3) ** 2, axis=0)
    h3 = g3_ref[...] * (h3 - m3) * lax.rsqrt(v3 + 1e-5) + be3_ref[...]
    logits = jnp.dot(h3, wo_ref[...], preferred_element_type=jnp.float32) + bo_ref[...]
    col = lax.broadcasted_iota(jnp.int32, logits.shape, 1)
    masked = jnp.where(col < 10, logits, -1e30)
    mx = jnp.max(masked, axis=1, keepdims=True)
    lse = jnp.log(jnp.sum(jnp.exp(masked - mx), axis=1, keepdims=True)) + mx
    out_ref[...] = masked - lse


def kernel(x, edge_index, batch, W1, b1, g1, be1, W2, b2, g2, be2, Wh, bh, g3, be3, Wo, bo):
    src3 = edge_index[0].reshape(NW, NCH, CH)
    dst3 = edge_index[1].reshape(NW, NCH, CH)
    ones16 = jnp.ones((CH, 16), jnp.float32)
    zeros16 = jnp.zeros((RPT, 16), jnp.float32)
    zerosD = jnp.zeros((RPT, D), jnp.float32)

    degp = _sc_degree(dst3, ones16, zeros16)

    y1, dinv = pl.pallas_call(
        _tc_first_body,
        out_shape=[jax.ShapeDtypeStruct((N, D), jnp.float32),
                   jax.ShapeDtypeStruct((N, 1), jnp.float32)],
    )(degp, x, W1)

    part1 = _sc_spmm(y1, src3, dst3, zerosD)

    y2 = pl.pallas_call(
        _tc_mid_body,
        out_shape=jax.ShapeDtypeStruct((N, D), jnp.float32),
    )(part1, y1, dinv, b1.reshape(1, D), g1.reshape(1, D), be1.reshape(1, D), W2)

    part2 = _sc_spmm(y2, src3, dst3, zerosD)

    out_pad = pl.pallas_call(
        _tc_head_body,
        out_shape=jax.ShapeDtypeStruct((N, D), jnp.float32),
    )(part2, y2, dinv, b2.reshape(1, D), g2.reshape(1, D), be2.reshape(1, D),
      Wh, bh.reshape(1, 64), g3.reshape(1, 64), be3.reshape(1, 64),
      jnp.pad(Wo, ((0, 0), (0, D - 10))), jnp.pad(bo, (0, D - 10)).reshape(1, D))
    return out_pad[:, :10]


# speedup vs baseline: 1.3535x; 1.2424x over previous
import functools

import jax
import jax.numpy as jnp
from jax import lax
from jax.experimental import pallas as pl
from jax.experimental.pallas import tpu as pltpu
from jax.experimental.pallas import tpu_sc as plsc

N = 10000
E = 320000
D = 128
NC = 2
NS = 16
NW = NC * NS
EPW = E // NW
CH = 80
NCH = EPW // CH
NP = 10112
RPT = NP // NS

_SC_MESH = plsc.VectorSubcoreMesh(core_axis_name="c", subcore_axis_name="s")


@functools.partial(
    pl.kernel,
    out_type=jax.ShapeDtypeStruct((NC, NP, 16), jnp.float32),
    mesh=_SC_MESH,
    scratch_types=[
        pltpu.VMEM((NCH, CH), jnp.int32),
        pltpu.VMEM((CH, 16), jnp.float32),
        pltpu.VMEM_SHARED((NP, 16), jnp.float32),
    ],
    compiler_params=pltpu.CompilerParams(use_tc_tiling_on_sc=False),
)
def _sc_degree(dst_hbm, ones_hbm, zeros_hbm, out_hbm, dst_v, ones_v, accum):
    c = lax.axis_index("c")
    s = lax.axis_index("s")
    w = s * NC + c
    pltpu.sync_copy(dst_hbm.at[w], dst_v)
    pltpu.sync_copy(ones_hbm, ones_v)
    pltpu.sync_copy(zeros_hbm, accum.at[pl.ds(s * RPT, RPT)])
    plsc.subcore_barrier()

    def body(j, carry):
        pltpu.sync_copy(ones_v, accum.at[dst_v.at[j]], add=True)
        return carry

    lax.fori_loop(0, NCH, body, 0)
    plsc.subcore_barrier()
    pltpu.sync_copy(accum.at[pl.ds(s * RPT, RPT)], out_hbm.at[c, pl.ds(s * RPT, RPT)])


@functools.partial(
    pl.kernel,
    out_type=jax.ShapeDtypeStruct((NC, NP, D), jnp.float32),
    mesh=_SC_MESH,
    scratch_types=[
        pltpu.VMEM((NCH, CH), jnp.int32),
        pltpu.VMEM((NCH, CH), jnp.int32),
        pltpu.VMEM((CH, D), jnp.float32),
        pltpu.VMEM((CH, D), jnp.float32),
        pltpu.VMEM_SHARED((NP, D), jnp.float32),
        pltpu.SemaphoreType.DMA,
    ],
    compiler_params=pltpu.CompilerParams(use_tc_tiling_on_sc=False),
)
def _sc_spmm(y_hbm, src_hbm, dst_hbm, zeros_hbm, out_hbm,
             src_v, dst_v, buf0, buf1, accum, sem):
    c = lax.axis_index("c")
    s = lax.axis_index("s")
    w = s * NC + c
    pltpu.sync_copy(src_hbm.at[w], src_v)
    pltpu.sync_copy(dst_hbm.at[w], dst_v)
    pltpu.sync_copy(zeros_hbm, accum.at[pl.ds(s * RPT, RPT)])
    plsc.subcore_barrier()

    pltpu.async_copy(y_hbm.at[src_v.at[0]], buf0, sem)

    def body(jj, carry):
        j0 = jj * 2
        pltpu.make_async_copy(y_hbm.at[src_v.at[j0]], buf0, sem).wait()
        pltpu.async_copy(y_hbm.at[src_v.at[j0 + 1]], buf1, sem)
        pltpu.sync_copy(buf0, accum.at[dst_v.at[j0]], add=True)
        pltpu.make_async_copy(y_hbm.at[src_v.at[j0 + 1]], buf1, sem).wait()

        @pl.when(jj + 1 < (NCH - 1) // 2)
        def _():
            pltpu.async_copy(y_hbm.at[src_v.at[j0 + 2]], buf0, sem)

        pltpu.sync_copy(buf1, accum.at[dst_v.at[j0 + 1]], add=True)
        return carry

    lax.fori_loop(0, (NCH - 1) // 2, body, 0)
    pltpu.async_copy(y_hbm.at[src_v.at[NCH - 1]], buf0, sem).wait()
    pltpu.sync_copy(buf0, accum.at[dst_v.at[NCH - 1]], add=True)
    plsc.subcore_barrier()
    pltpu.sync_copy(accum.at[pl.ds(s * RPT, RPT)], out_hbm.at[c, pl.ds(s * RPT, RPT)])


def _tc_first_body(degp_ref, x_ref, w_ref, y_ref, dinv_ref):
    indeg = degp_ref[0, :N, 0:1] + degp_ref[1, :N, 0:1]
    dinv = lax.rsqrt(indeg + 1.0)
    xw = jnp.dot(x_ref[...], w_ref[...], preferred_element_type=jnp.float32)
    y_ref[...] = xw * dinv
    dinv_ref[...] = dinv


def _tc_mid_body(part_ref, y_ref, dinv_ref, b_ref, g_ref, be_ref, w_ref, out_ref):
    dinv = dinv_ref[...]
    h = (part_ref[0, :N] + part_ref[1, :N] + y_ref[...]) * dinv + b_ref[...]
    m = jnp.mean(h, axis=0)
    v = jnp.mean((h - m) ** 2, axis=0)
    h = jnp.maximum(g_ref[...] * (h - m) * lax.rsqrt(v + 1e-5) + be_ref[...], 0.0)
    out_ref[...] = jnp.dot(h, w_ref[...], preferred_element_type=jnp.float32) * dinv


def _tc_head_body(part_ref, y_ref, dinv_ref, b_ref, g_ref, be_ref,
                  wh_ref, bh_ref, g3_ref, be3_ref, wo_ref, bo_ref, out_ref):
    dinv = dinv_ref[...]
    h = (part_ref[0, :N] + part_ref[1, :N] + y_ref[...]) * dinv + b_ref[...]
    m = jnp.mean(h, axis=0)
    v = jnp.mean((h - m) ** 2, axis=0)
    h = jnp.maximum(g_ref[...] * (h - m) * lax.rsqrt(v + 1e-5) + be_ref[...], 0.0)
    h3 = jnp.maximum(
        jnp.dot(h, wh_ref[...], preferred_element_type=jnp.float32) + bh_ref[...], 0.0)
    m3 = jnp.mean(h3, axis=0)
    v3 = jnp.mean((h3 - m3) ** 2, axis=0)
    h3 = g3_ref[...] * (h3 - m3) * lax.rsqrt(v3 + 1e-5) + be3_ref[...]
    logits = jnp.dot(h3, wo_ref[...], preferred_element_type=jnp.float32) + bo_ref[...]
    col = lax.broadcasted_iota(jnp.int32, logits.shape, 1)
    masked = jnp.where(col < 10, logits, -1e30)
    mx = jnp.max(masked, axis=1, keepdims=True)
    lse = jnp.log(jnp.sum(jnp.exp(masked - mx), axis=1, keepdims=True)) + mx
    out_ref[...] = masked - lse


def kernel(x, edge_index, batch, W1, b1, g1, be1, W2, b2, g2, be2, Wh, bh, g3, be3, Wo, bo):
    src3 = edge_index[0].reshape(NW, NCH, CH)
    dst3 = edge_index[1].reshape(NW, NCH, CH)
    ones16 = jnp.ones((CH, 16), jnp.float32)
    zeros16 = jnp.zeros((RPT, 16), jnp.float32)
    zerosD = jnp.zeros((RPT, D), jnp.float32)

    degp = _sc_degree(dst3, ones16, zeros16)

    y1, dinv = pl.pallas_call(
        _tc_first_body,
        out_shape=[jax.ShapeDtypeStruct((N, D), jnp.float32),
                   jax.ShapeDtypeStruct((N, 1), jnp.float32)],
    )(degp, x, W1)

    part1 = _sc_spmm(y1, src3, dst3, zerosD)

    y2 = pl.pallas_call(
        _tc_mid_body,
        out_shape=jax.ShapeDtypeStruct((N, D), jnp.float32),
    )(part1, y1, dinv, b1.reshape(1, D), g1.reshape(1, D), be1.reshape(1, D), W2)

    part2 = _sc_spmm(y2, src3, dst3, zerosD)

    out_pad = pl.pallas_call(
        _tc_head_body,
        out_shape=jax.ShapeDtypeStruct((N, D), jnp.float32),
    )(part2, y2, dinv, b2.reshape(1, D), g2.reshape(1, D), be2.reshape(1, D),
      Wh, bh.reshape(1, 64), g3.reshape(1, 64), be3.reshape(1, 64),
      jnp.pad(Wo, ((0, 0), (0, D - 10))), jnp.pad(bo, (0, D - 10)).reshape(1, D))
    return out_pad[:, :10]


# speedup vs baseline: 1.8607x; 1.3748x over previous
import functools

import jax
import jax.numpy as jnp
from jax import lax
from jax.experimental import pallas as pl
from jax.experimental.pallas import tpu as pltpu
from jax.experimental.pallas import tpu_sc as plsc

N = 10000
E = 320000
D = 128
NC = 2
NS = 16
NW = NC * NS
EPW = E // NW
CH = 80
NCH = EPW // CH
NP = 10000
RPT = NP // NS

_SC_MESH = plsc.VectorSubcoreMesh(core_axis_name="c", subcore_axis_name="s")


@functools.partial(
    pl.kernel,
    out_type=jax.ShapeDtypeStruct((NC, NP, 16), jnp.float32),
    mesh=_SC_MESH,
    scratch_types=[
        pltpu.VMEM((NCH, CH), jnp.int32),
        pltpu.VMEM((CH, 16), jnp.float32),
        pltpu.VMEM_SHARED((NP, 16), jnp.float32),
    ],
    compiler_params=pltpu.CompilerParams(use_tc_tiling_on_sc=False),
)
def _sc_degree(dst_hbm, ones_hbm, zeros_hbm, out_hbm, dst_v, ones_v, accum):
    c = lax.axis_index("c")
    s = lax.axis_index("s")
    w = s * NC + c
    pltpu.sync_copy(dst_hbm.at[w], dst_v)
    pltpu.sync_copy(ones_hbm, ones_v)
    pltpu.sync_copy(zeros_hbm, accum.at[pl.ds(s * RPT, RPT)])
    plsc.subcore_barrier()

    def body(j, carry):
        pltpu.sync_copy(ones_v, accum.at[dst_v.at[j]], add=True)
        return carry

    lax.fori_loop(0, NCH, body, 0)
    plsc.subcore_barrier()
    pltpu.sync_copy(accum.at[pl.ds(s * RPT, RPT)], out_hbm.at[c, pl.ds(s * RPT, RPT)])


@functools.partial(
    pl.kernel,
    out_type=jax.ShapeDtypeStruct((NC, NP, D), jnp.float32),
    mesh=_SC_MESH,
    scratch_types=[
        pltpu.VMEM((NCH, CH), jnp.int32),
        pltpu.VMEM((NCH, CH), jnp.int32),
        pltpu.VMEM((CH, D), jnp.float32),
        pltpu.VMEM((CH, D), jnp.float32),
        pltpu.VMEM((CH, D), jnp.float32),
        pltpu.VMEM_SHARED((NP, D), jnp.float32),
        pltpu.SemaphoreType.DMA,
    ],
    compiler_params=pltpu.CompilerParams(use_tc_tiling_on_sc=False),
)
def _sc_spmm(y_hbm, src_hbm, dst_hbm, zeros_hbm, out_hbm,
             src_v, dst_v, buf0, buf1, buf2, accum, sem):
    c = lax.axis_index("c")
    s = lax.axis_index("s")
    w = s * NC + c
    pltpu.sync_copy(src_hbm.at[w], src_v)
    pltpu.sync_copy(dst_hbm.at[w], dst_v)
    pltpu.sync_copy(zeros_hbm, accum.at[pl.ds(s * RPT, RPT)])
    plsc.subcore_barrier()

    bufs = (buf0, buf1, buf2)
    pltpu.async_copy(y_hbm.at[src_v.at[0]], buf0, sem)
    pltpu.async_copy(y_hbm.at[src_v.at[1]], buf1, sem)

    def body(jj, carry):
        j0 = jj * 3
        for k in range(3):
            j = j0 + k
            pltpu.make_async_copy(y_hbm.at[src_v.at[j]], bufs[k], sem).wait()

            @pl.when(j + 2 < NCH)
            def _():
                pltpu.async_copy(
                    y_hbm.at[src_v.at[j + 2]], bufs[(k + 2) % 3], sem)

            pltpu.sync_copy(bufs[k], accum.at[dst_v.at[j]], add=True)
        return carry

    lax.fori_loop(0, NCH // 3, body, 0)
    pltpu.make_async_copy(y_hbm.at[src_v.at[NCH - 2]], buf0, sem).wait()
    pltpu.sync_copy(buf0, accum.at[dst_v.at[NCH - 2]], add=True)
    pltpu.make_async_copy(y_hbm.at[src_v.at[NCH - 1]], buf1, sem).wait()
    pltpu.sync_copy(buf1, accum.at[dst_v.at[NCH - 1]], add=True)
    plsc.subcore_barrier()
    pltpu.sync_copy(accum.at[pl.ds(s * RPT, RPT)], out_hbm.at[c, pl.ds(s * RPT, RPT)])


def _tc_first_body(degp_ref, x_ref, w_ref, y_ref, dinv_ref):
    indeg = degp_ref[0, :N, 0:1] + degp_ref[1, :N, 0:1]
    dinv = lax.rsqrt(indeg + 1.0)
    xw = jnp.dot(x_ref[...], w_ref[...], preferred_element_type=jnp.float32)
    y_ref[...] = xw * dinv
    dinv_ref[...] = dinv


def _tc_mid_body(part_ref, y_ref, dinv_ref, b_ref, g_ref, be_ref, w_ref, out_ref):
    dinv = dinv_ref[...]
    h = (part_ref[0, :N] + part_ref[1, :N] + y_ref[...]) * dinv + b_ref[...]
    m = jnp.mean(h, axis=0)
    v = jnp.mean((h - m) ** 2, axis=0)
    h = jnp.maximum(g_ref[...] * (h - m) * lax.rsqrt(v + 1e-5) + be_ref[...], 0.0)
    out_ref[...] = jnp.dot(h, w_ref[...], preferred_element_type=jnp.float32) * dinv


def _tc_head_body(part_ref, y_ref, dinv_ref, b_ref, g_ref, be_ref,
                  wh_ref, bh_ref, g3_ref, be3_ref, wo_ref, bo_ref, out_ref):
    dinv = dinv_ref[...]
    h = (part_ref[0, :N] + part_ref[1, :N] + y_ref[...]) * dinv + b_ref[...]
    m = jnp.mean(h, axis=0)
    v = jnp.mean((h - m) ** 2, axis=0)
    h = jnp.maximum(g_ref[...] * (h - m) * lax.rsqrt(v + 1e-5) + be_ref[...], 0.0)
    h3 = jnp.maximum(
        jnp.dot(h, wh_ref[...], preferred_element_type=jnp.float32) + bh_ref[...], 0.0)
    m3 = jnp.mean(h3, axis=0)
    v3 = jnp.mean((h3 - m3) ** 2, axis=0)
    h3 = g3_ref[...] * (h3 - m3) * lax.rsqrt(v3 + 1e-5) + be3_ref[...]
    logits = jnp.dot(h3, wo_ref[...], preferred_element_type=jnp.float32) + bo_ref[...]
    col = lax.broadcasted_iota(jnp.int32, logits.shape, 1)
    masked = jnp.where(col < 10, logits, -1e30)
    mx = jnp.max(masked, axis=1, keepdims=True)
    lse = jnp.log(jnp.sum(jnp.exp(masked - mx), axis=1, keepdims=True)) + mx
    out_ref[...] = masked - lse


def kernel(x, edge_index, batch, W1, b1, g1, be1, W2, b2, g2, be2, Wh, bh, g3, be3, Wo, bo):
    src3 = edge_index[0].reshape(NW, NCH, CH)
    dst3 = edge_index[1].reshape(NW, NCH, CH)
    ones16 = jnp.ones((CH, 16), jnp.float32)
    zeros16 = jnp.zeros((RPT, 16), jnp.float32)
    zerosD = jnp.zeros((RPT, D), jnp.float32)

    degp = _sc_degree(dst3, ones16, zeros16)

    y1, dinv = pl.pallas_call(
        _tc_first_body,
        out_shape=[jax.ShapeDtypeStruct((N, D), jnp.float32),
                   jax.ShapeDtypeStruct((N, 1), jnp.float32)],
    )(degp, x, W1)

    part1 = _sc_spmm(y1, src3, dst3, zerosD)

    y2 = pl.pallas_call(
        _tc_mid_body,
        out_shape=jax.ShapeDtypeStruct((N, D), jnp.float32),
    )(part1, y1, dinv, b1.reshape(1, D), g1.reshape(1, D), be1.reshape(1, D), W2)

    part2 = _sc_spmm(y2, src3, dst3, zerosD)

    out_pad = pl.pallas_call(
        _tc_head_body,
        out_shape=jax.ShapeDtypeStruct((N, D), jnp.float32),
    )(part2, y2, dinv, b2.reshape(1, D), g2.reshape(1, D), be2.reshape(1, D),
      Wh, bh.reshape(1, 64), g3.reshape(1, 64), be3.reshape(1, 64),
      jnp.pad(Wo, ((0, 0), (0, D - 10))), jnp.pad(bo, (0, D - 10)).reshape(1, D))
    return out_pad[:, :10]
